# baseline (speedup 1.0000x reference)
import jax
import jax.numpy as jnp
from jax.experimental import pallas as pl
from jax.experimental.pallas import tpu as pltpu
from jax.experimental.pallas import tpu_sc as plsc

G = 16
B_TOT = 256
K = 1024
D = 64
HID = 128

NC, NS = 2, 16
NW = NC * NS
ROWS = B_TOT * 64
BPW = ROWS // NW


def _mm(a, b):
    return jax.lax.dot_general(a, b, (((1,), (0,)), ((), ())),
                               preferred_element_type=jnp.float32)


def _enc_block(p1_ref, w1_ref, b1_ref, w2c_ref, b2_ref, cbt_ref, idx_ref):
    g = p1_ref.shape[0]
    f32 = jnp.float32

    p1 = p1_ref[...]
    m = _mm(p1.reshape(g * 4 * 64, 48), w1_ref[...])
    h1 = jax.nn.relu(m + b1_ref[...][None, :]).reshape(g, 4, 8, 8, HID)

    h1p = {}
    for py in range(2):
        for px in range(2):
            h1p[(py, px)] = jnp.pad(
                h1[:, py * 2 + px],
                ((0, 0), (py, 1 - py), (px, 1 - px), (0, 0)))

    w2c = w2c_ref[...]
    e = jnp.zeros((g * 64, D), f32)
    for pr in range(2):
        for pc in range(2):
            a = h1p[(pr, pc)]
            cat = jnp.concatenate(
                [a[:, oy:oy + 8, ox:ox + 8, :]
                 for oy in range(2) for ox in range(2)], axis=-1)
            e = e + _mm(cat.reshape(g * 64, 4 * HID), w2c[pr * 2 + pc])
    e = e + b2_ref[...][None, :]

    cbt = cbt_ref[...]
    cn = jnp.sum(cbt * cbt, axis=0)
    scores = cn[None, :] - 2.0 * _mm(e, cbt)
    idx_ref[...] = jnp.argmin(scores, axis=1).astype(jnp.int32).reshape(g, 64)


def _sc_gather_body(table_ref, idx_ref, out_ref, idx_v, rows_v, sem):
    wid = jax.lax.axis_index("s") * NC + jax.lax.axis_index("c")
    base = wid * BPW
    pltpu.sync_copy(idx_ref.at[pl.ds(base, BPW)], idx_v)
    pltpu.async_copy(table_ref.at[idx_v], rows_v, sem).wait()
    pltpu.sync_copy(rows_v, out_ref.at[pl.ds(base, BPW)])


def _dec_block(q_ref, d1t_ref, db1_ref, d2a_ref, db2_ref, out_ref):
    g = q_ref.shape[0] // 64
    q = q_ref[...]

    d1t = d1t_ref[...]
    db1 = db1_ref[...]
    big = _mm(q, d1t)
    pp1 = [jnp.pad(big[:, HID * t:HID * (t + 1)].reshape(g, 8, 8, HID),
                   ((0, 0), (1, 1), (1, 1), (0, 0)))
           for t in range(16)]
    h2 = {}
    for R in range(2):
        for S in range(2):
            acc = None
            for i in range(2):
                sy = R - 1 + i
                for j in range(2):
                    sx = S - 1 + j
                    t = (R + 2 * i) * 4 + (S + 2 * j)
                    sl = pp1[t][:, 1 + sy:9 + sy, 1 + sx:9 + sx, :]
                    acc = sl if acc is None else acc + sl
            h2[(R, S)] = jax.nn.relu(acc + db1[None, None, None, :])

    d2c = d2a_ref[...]
    db2 = db2_ref[...]
    h2p = {k: jnp.pad(v, ((0, 0), (1, 1), (1, 1), (0, 0)))
           for k, v in h2.items()}
    acc48 = None
    for ty in range(4):
        R, apar = ty & 1, ty >> 1
        for tx in range(4):
            S, bpar = tx & 1, tx >> 1
            parts = []
            for i in range(2):
                cy = apar + R - 1 + i
                ry, sy = cy & 1, cy >> 1
                for j in range(2):
                    cx = bpar + S - 1 + j
                    rx, sx = cx & 1, cx >> 1
                    parts.append(h2p[(ry, rx)][:, 1 + sy:9 + sy,
                                               1 + sx:9 + sx, :])
            cat = jnp.concatenate(parts, axis=-1)
            res = _mm(cat.reshape(g * 64, 4 * HID), d2c[ty * 4 + tx])
            acc48 = res if acc48 is None else acc48 + res
    out_ref[...] = (acc48 + db2[None, :]).reshape(g, 8, 8, 48)


def _run_enc(p1p, w1r, b1, w2c, b2, cbt, *, interpret=False):
    grid = (B_TOT // G,)
    full = lambda a: pl.BlockSpec(a.shape, lambda i: (0,) * a.ndim)
    return pl.pallas_call(
        _enc_block,
        grid=grid,
        in_specs=[pl.BlockSpec((G, 4, 8, 8, 48), lambda i: (i, 0, 0, 0, 0)),
                  full(w1r), full(b1), full(w2c), full(b2), full(cbt)],
        out_specs=pl.BlockSpec((G, 64), lambda i: (i, 0)),
        out_shape=jax.ShapeDtypeStruct((B_TOT, 64), jnp.int32),
        interpret=interpret,
    )(p1p, w1r, b1, w2c, b2, cbt)


def _run_gather(cb128, idx_flat):
    mesh = plsc.VectorSubcoreMesh(core_axis_name="c", subcore_axis_name="s")
    return pl.kernel(
        _sc_gather_body,
        out_type=jax.ShapeDtypeStruct((ROWS, 2 * D), jnp.float32),
        mesh=mesh,
        scratch_types=[
            pltpu.VMEM((BPW,), jnp.int32),
            pltpu.VMEM((BPW, 2 * D), jnp.float32),
            pltpu.SemaphoreType.DMA,
        ],
    )(cb128, idx_flat)


def _run_dec(q, d1t, db1, d2a, db2t, *, interpret=False):
    grid = (B_TOT // G,)
    full = lambda a: pl.BlockSpec(a.shape, lambda i: (0,) * a.ndim)
    return pl.pallas_call(
        _dec_block,
        grid=grid,
        in_specs=[pl.BlockSpec((G * 64, 2 * D), lambda i: (i, 0)),
                  full(d1t), full(db1), full(d2a), full(db2t)],
        out_specs=pl.BlockSpec((G, 8, 8, 48), lambda i: (i, 0, 0, 0)),
        out_shape=jax.ShapeDtypeStruct((B_TOT, 8, 8, 48), jnp.float32),
        interpret=interpret,
    )(q, d1t, db1, d2a, db2t)


def kernel(x, enc_w1, enc_b1, enc_w2, enc_b2, codebook,
           dec_w1, dec_b1, dec_w2, dec_b2):
    xn = jnp.transpose(x, (0, 2, 3, 1))
    xp = jnp.pad(xn, ((0, 0), (1, 1), (1, 1), (0, 0)))
    p1p = jnp.stack(
        [jnp.concatenate(
            [xp[:, 2 * py + kh:2 * py + kh + 29:4,
                2 * px + kw:2 * px + kw + 29:4, :]
             for kh in range(4) for kw in range(4)], axis=-1)
         for py in range(2) for px in range(2)], axis=1)

    w1r = enc_w1.transpose(2, 3, 1, 0).reshape(48, HID)
    w2k = enc_w2.transpose(2, 3, 1, 0)
    w2c = jnp.stack(
        [jnp.concatenate(
            [w2k[(1 - pr) + 2 * oy, (1 - pc) + 2 * ox]
             for oy in range(2) for ox in range(2)], axis=0)
         for pr in range(2) for pc in range(2)], axis=0)
    d1t = jnp.pad(dec_w1.transpose(1, 2, 3, 0).reshape(D, 16 * HID),
                  ((0, D), (0, 0)))
    d2k = dec_w2.transpose(2, 3, 1, 0)
    d2a = jnp.stack(
        [jnp.pad(
            jnp.concatenate(
                [d2k[(ty & 1) + 2 * i, (tx & 1) + 2 * j]
                 for i in range(2) for j in range(2)], axis=0),
            ((0, 0), (3 * (ty * 4 + tx), 45 - 3 * (ty * 4 + tx))))
         for ty in range(4) for tx in range(4)], axis=0)
    db2t = jnp.tile(dec_b2, 16)

    cb128 = jnp.pad(codebook, ((0, 0), (0, D)))
    idx = _run_enc(p1p, w1r, enc_b1, w2c, enc_b2, codebook.T)
    q = _run_gather(cb128, idx.reshape(ROWS))
    res48 = _run_dec(q, d1t, dec_b1, d2a, db2t)

    st = res48.reshape(B_TOT, 8, 8, 4, 4, 3)
    return st.transpose(0, 5, 1, 3, 2, 4).reshape(B_TOT, 3, 32, 32)


# speedup vs baseline: 1.0013x; 1.0013x over previous
import jax
import jax.numpy as jnp
from jax.experimental import pallas as pl
from jax.experimental.pallas import tpu as pltpu
from jax.experimental.pallas import tpu_sc as plsc

G = 16
B_TOT = 256
K = 1024
D = 64
HID = 128

NC, NS = 2, 16
NW = NC * NS
ROWS = B_TOT * 64
BPW = ROWS // NW


def _mm(a, b):
    return jax.lax.dot_general(a, b, (((1,), (0,)), ((), ())),
                               preferred_element_type=jnp.float32)


def _enc_block(p1_ref, w1_ref, b1_ref, w2c_ref, b2_ref, cbt_ref, idx_ref):
    g = p1_ref.shape[0]
    f32 = jnp.float32

    p1 = p1_ref[...]
    m = _mm(p1.reshape(g * 4 * 64, 48), w1_ref[...])
    h1 = jax.nn.relu(m + b1_ref[...][None, :]).reshape(g, 4, 8, 8, HID)

    h1p = {}
    for py in range(2):
        for px in range(2):
            h1p[(py, px)] = jnp.pad(
                h1[:, py * 2 + px],
                ((0, 0), (py, 1 - py), (px, 1 - px), (0, 0)))

    w2c = w2c_ref[...]
    e = jnp.zeros((g * 64, D), f32)
    for pr in range(2):
        for pc in range(2):
            a = h1p[(pr, pc)]
            cat = jnp.concatenate(
                [a[:, oy:oy + 8, ox:ox + 8, :]
                 for oy in range(2) for ox in range(2)], axis=-1)
            e = e + _mm(cat.reshape(g * 64, 4 * HID), w2c[pr * 2 + pc])
    e = e + b2_ref[...][None, :]

    cbt = cbt_ref[...]
    cn = jnp.sum(cbt * cbt, axis=0)
    scores = cn[None, :] - 2.0 * _mm(e, cbt)
    idx_ref[...] = jnp.argmin(scores, axis=1).astype(jnp.int32).reshape(g, 64)


SC_CH = 8
SC_SZ = BPW // SC_CH


def _sc_gather_body(table_ref, idx_ref, out_ref, idx_v, rows_v, *sems):
    wid = jax.lax.axis_index("s") * NC + jax.lax.axis_index("c")
    base = wid * BPW
    pltpu.sync_copy(idx_ref.at[pl.ds(base, BPW)], idx_v)
    copies = [
        pltpu.async_copy(table_ref.at[idx_v.at[pl.ds(c * SC_SZ, SC_SZ)]],
                         rows_v.at[pl.ds(c * SC_SZ, SC_SZ)], sems[c])
        for c in range(SC_CH)]
    for cp in copies:
        cp.wait()
    pltpu.sync_copy(rows_v, out_ref.at[pl.ds(base, BPW)])


def _dec_block(q_ref, d1t_ref, db1_ref, d2a_ref, db2_ref, out_ref):
    g = q_ref.shape[0] // 64
    q = q_ref[...]

    d1t = d1t_ref[...]
    db1 = db1_ref[...]
    big = _mm(q, d1t)
    pp1 = [jnp.pad(big[:, HID * t:HID * (t + 1)].reshape(g, 8, 8, HID),
                   ((0, 0), (1, 1), (1, 1), (0, 0)))
           for t in range(16)]
    h2 = {}
    for R in range(2):
        for S in range(2):
            acc = None
            for i in range(2):
                sy = R - 1 + i
                for j in range(2):
                    sx = S - 1 + j
                    t = (R + 2 * i) * 4 + (S + 2 * j)
                    sl = pp1[t][:, 1 + sy:9 + sy, 1 + sx:9 + sx, :]
                    acc = sl if acc is None else acc + sl
            h2[(R, S)] = jax.nn.relu(acc + db1[None, None, None, :])

    d2c = d2a_ref[...]
    db2 = db2_ref[...]
    h2p = {k: jnp.pad(v, ((0, 0), (1, 1), (1, 1), (0, 0)))
           for k, v in h2.items()}
    acc48 = None
    for ty in range(4):
        R, apar = ty & 1, ty >> 1
        for tx in range(4):
            S, bpar = tx & 1, tx >> 1
            parts = []
            for i in range(2):
                cy = apar + R - 1 + i
                ry, sy = cy & 1, cy >> 1
                for j in range(2):
                    cx = bpar + S - 1 + j
                    rx, sx = cx & 1, cx >> 1
                    parts.append(h2p[(ry, rx)][:, 1 + sy:9 + sy,
                                               1 + sx:9 + sx, :])
            cat = jnp.concatenate(parts, axis=-1)
            res = _mm(cat.reshape(g * 64, 4 * HID), d2c[ty * 4 + tx])
            acc48 = res if acc48 is None else acc48 + res
    out_ref[...] = (acc48 + db2[None, :]).reshape(g, 8, 8, 48)


def _run_enc(p1p, w1r, b1, w2c, b2, cbt, *, interpret=False):
    grid = (B_TOT // G,)
    full = lambda a: pl.BlockSpec(a.shape, lambda i: (0,) * a.ndim)
    return pl.pallas_call(
        _enc_block,
        grid=grid,
        in_specs=[pl.BlockSpec((G, 4, 8, 8, 48), lambda i: (i, 0, 0, 0, 0)),
                  full(w1r), full(b1), full(w2c), full(b2), full(cbt)],
        out_specs=pl.BlockSpec((G, 64), lambda i: (i, 0)),
        out_shape=jax.ShapeDtypeStruct((B_TOT, 64), jnp.int32),
        interpret=interpret,
    )(p1p, w1r, b1, w2c, b2, cbt)


def _run_gather(cb128, idx_flat):
    mesh = plsc.VectorSubcoreMesh(core_axis_name="c", subcore_axis_name="s")
    return pl.kernel(
        _sc_gather_body,
        out_type=jax.ShapeDtypeStruct((ROWS, 2 * D), jnp.float32),
        mesh=mesh,
        scratch_types=(
            [pltpu.VMEM((BPW,), jnp.int32),
             pltpu.VMEM((BPW, 2 * D), jnp.float32)]
            + [pltpu.SemaphoreType.DMA] * SC_CH),
    )(cb128, idx_flat)


def _run_dec(q, d1t, db1, d2a, db2t, *, interpret=False):
    grid = (B_TOT // G,)
    full = lambda a: pl.BlockSpec(a.shape, lambda i: (0,) * a.ndim)
    return pl.pallas_call(
        _dec_block,
        grid=grid,
        in_specs=[pl.BlockSpec((G * 64, 2 * D), lambda i: (i, 0)),
                  full(d1t), full(db1), full(d2a), full(db2t)],
        out_specs=pl.BlockSpec((G, 8, 8, 48), lambda i: (i, 0, 0, 0)),
        out_shape=jax.ShapeDtypeStruct((B_TOT, 8, 8, 48), jnp.float32),
        interpret=interpret,
    )(q, d1t, db1, d2a, db2t)


def kernel(x, enc_w1, enc_b1, enc_w2, enc_b2, codebook,
           dec_w1, dec_b1, dec_w2, dec_b2):
    xn = jnp.transpose(x, (0, 2, 3, 1))
    xp = jnp.pad(xn, ((0, 0), (1, 1), (1, 1), (0, 0)))
    p1p = jnp.stack(
        [jnp.concatenate(
            [xp[:, 2 * py + kh:2 * py + kh + 29:4,
                2 * px + kw:2 * px + kw + 29:4, :]
             for kh in range(4) for kw in range(4)], axis=-1)
         for py in range(2) for px in range(2)], axis=1)

    w1r = enc_w1.transpose(2, 3, 1, 0).reshape(48, HID)
    w2k = enc_w2.transpose(2, 3, 1, 0)
    w2c = jnp.stack(
        [jnp.concatenate(
            [w2k[(1 - pr) + 2 * oy, (1 - pc) + 2 * ox]
             for oy in range(2) for ox in range(2)], axis=0)
         for pr in range(2) for pc in range(2)], axis=0)
    d1t = jnp.pad(dec_w1.transpose(1, 2, 3, 0).reshape(D, 16 * HID),
                  ((0, D), (0, 0)))
    d2k = dec_w2.transpose(2, 3, 1, 0)
    d2a = jnp.stack(
        [jnp.pad(
            jnp.concatenate(
                [d2k[(ty & 1) + 2 * i, (tx & 1) + 2 * j]
                 for i in range(2) for j in range(2)], axis=0),
            ((0, 0), (3 * (ty * 4 + tx), 45 - 3 * (ty * 4 + tx))))
         for ty in range(4) for tx in range(4)], axis=0)
    db2t = jnp.tile(dec_b2, 16)

    cb128 = jnp.pad(codebook, ((0, 0), (0, D)))
    idx = _run_enc(p1p, w1r, enc_b1, w2c, enc_b2, codebook.T)
    q = _run_gather(cb128, idx.reshape(ROWS))
    res48 = _run_dec(q, d1t, dec_b1, d2a, db2t)

    st = res48.reshape(B_TOT, 8, 8, 4, 4, 3)
    return st.transpose(0, 5, 1, 3, 2, 4).reshape(B_TOT, 3, 32, 32)


# speedup vs baseline: 1.8833x; 1.8809x over previous
import jax
import jax.numpy as jnp
from jax.experimental import pallas as pl
from jax.experimental.pallas import tpu as pltpu
from jax.experimental.pallas import tpu_sc as plsc

G = 16
B_TOT = 256
K = 1024
D = 64
HID = 128

NC, NS = 2, 16
NW = NC * NS
ROWS = B_TOT * 64
BPW = ROWS // NW


def _mm(a, b):
    return jax.lax.dot_general(a, b, (((1,), (0,)), ((), ())),
                               preferred_element_type=jnp.float32)


def _enc_block(p1_ref, w1_ref, b1_ref, w2c_ref, b2_ref, cbt_ref, idx_ref):
    g = p1_ref.shape[0]
    f32 = jnp.float32

    p1 = p1_ref[...]
    m = _mm(p1.reshape(g * 4 * 64, 48), w1_ref[...])
    h1 = jax.nn.relu(m + b1_ref[...][None, :]).reshape(g, 4, 8, 8, HID)

    h1p = {}
    for py in range(2):
        for px in range(2):
            h1p[(py, px)] = jnp.pad(
                h1[:, py * 2 + px],
                ((0, 0), (py, 1 - py), (px, 1 - px), (0, 0)))

    w2c = w2c_ref[...]
    e = jnp.zeros((g * 64, D), f32)
    for pr in range(2):
        for pc in range(2):
            a = h1p[(pr, pc)]
            cat = jnp.concatenate(
                [a[:, oy:oy + 8, ox:ox + 8, :]
                 for oy in range(2) for ox in range(2)], axis=-1)
            e = e + _mm(cat.reshape(g * 64, 4 * HID), w2c[pr * 2 + pc])
    e = e + b2_ref[...][None, :]

    cbt = cbt_ref[...]
    cn = jnp.sum(cbt * cbt, axis=0)
    scores = cn[None, :] - 2.0 * _mm(e, cbt)
    idx_ref[...] = jnp.argmin(scores, axis=1).astype(jnp.int32).reshape(g, 64)


SC_STAGE = K // NS


def _sc_gather_body(table_ref, idx_ref, out_ref, tspm, stage_v, idx_v,
                    rows_v, sem):
    sid = jax.lax.axis_index("s")
    wid = sid * NC + jax.lax.axis_index("c")
    pltpu.sync_copy(table_ref.at[pl.ds(sid * SC_STAGE, SC_STAGE)], stage_v)
    pltpu.sync_copy(stage_v, tspm.at[pl.ds(sid * SC_STAGE, SC_STAGE)])
    plsc.subcore_barrier()
    base = wid * BPW
    pltpu.sync_copy(idx_ref.at[pl.ds(base, BPW)], idx_v)
    pltpu.async_copy(tspm.at[idx_v], rows_v, sem).wait()
    pltpu.sync_copy(rows_v, out_ref.at[pl.ds(base, BPW)])


def _dec_block(q_ref, d1t_ref, db1_ref, d2a_ref, db2_ref, out_ref):
    g = q_ref.shape[0] // 64
    q = q_ref[...]

    d1t = d1t_ref[...]
    db1 = db1_ref[...]
    big = _mm(q, d1t)
    pp1 = [jnp.pad(big[:, HID * t:HID * (t + 1)].reshape(g, 8, 8, HID),
                   ((0, 0), (1, 1), (1, 1), (0, 0)))
           for t in range(16)]
    h2 = {}
    for R in range(2):
        for S in range(2):
            acc = None
            for i in range(2):
                sy = R - 1 + i
                for j in range(2):
                    sx = S - 1 + j
                    t = (R + 2 * i) * 4 + (S + 2 * j)
                    sl = pp1[t][:, 1 + sy:9 + sy, 1 + sx:9 + sx, :]
                    acc = sl if acc is None else acc + sl
            h2[(R, S)] = jax.nn.relu(acc + db1[None, None, None, :])

    d2c = d2a_ref[...]
    db2 = db2_ref[...]
    h2p = {k: jnp.pad(v, ((0, 0), (1, 1), (1, 1), (0, 0)))
           for k, v in h2.items()}
    acc48 = None
    for ty in range(4):
        R, apar = ty & 1, ty >> 1
        for tx in range(4):
            S, bpar = tx & 1, tx >> 1
            parts = []
            for i in range(2):
                cy = apar + R - 1 + i
                ry, sy = cy & 1, cy >> 1
                for j in range(2):
                    cx = bpar + S - 1 + j
                    rx, sx = cx & 1, cx >> 1
                    parts.append(h2p[(ry, rx)][:, 1 + sy:9 + sy,
                                               1 + sx:9 + sx, :])
            cat = jnp.concatenate(parts, axis=-1)
            res = _mm(cat.reshape(g * 64, 4 * HID), d2c[ty * 4 + tx])
            acc48 = res if acc48 is None else acc48 + res
    out_ref[...] = (acc48 + db2[None, :]).reshape(g, 8, 8, 48)


def _run_enc(p1p, w1r, b1, w2c, b2, cbt, *, interpret=False):
    grid = (B_TOT // G,)
    full = lambda a: pl.BlockSpec(a.shape, lambda i: (0,) * a.ndim)
    return pl.pallas_call(
        _enc_block,
        grid=grid,
        in_specs=[pl.BlockSpec((G, 4, 8, 8, 48), lambda i: (i, 0, 0, 0, 0)),
                  full(w1r), full(b1), full(w2c), full(b2), full(cbt)],
        out_specs=pl.BlockSpec((G, 64), lambda i: (i, 0)),
        out_shape=jax.ShapeDtypeStruct((B_TOT, 64), jnp.int32),
        interpret=interpret,
    )(p1p, w1r, b1, w2c, b2, cbt)


def _run_gather(cb128, idx_flat):
    mesh = plsc.VectorSubcoreMesh(core_axis_name="c", subcore_axis_name="s")
    return pl.kernel(
        _sc_gather_body,
        out_type=jax.ShapeDtypeStruct((ROWS, 2 * D), jnp.float32),
        mesh=mesh,
        scratch_types=[
            pltpu.VMEM_SHARED((K, 2 * D), jnp.float32),
            pltpu.VMEM((SC_STAGE, 2 * D), jnp.float32),
            pltpu.VMEM((BPW,), jnp.int32),
            pltpu.VMEM((BPW, 2 * D), jnp.float32),
            pltpu.SemaphoreType.DMA,
        ],
    )(cb128, idx_flat)


def _run_dec(q, d1t, db1, d2a, db2t, *, interpret=False):
    grid = (B_TOT // G,)
    full = lambda a: pl.BlockSpec(a.shape, lambda i: (0,) * a.ndim)
    return pl.pallas_call(
        _dec_block,
        grid=grid,
        in_specs=[pl.BlockSpec((G * 64, 2 * D), lambda i: (i, 0)),
                  full(d1t), full(db1), full(d2a), full(db2t)],
        out_specs=pl.BlockSpec((G, 8, 8, 48), lambda i: (i, 0, 0, 0)),
        out_shape=jax.ShapeDtypeStruct((B_TOT, 8, 8, 48), jnp.float32),
        interpret=interpret,
    )(q, d1t, db1, d2a, db2t)


def kernel(x, enc_w1, enc_b1, enc_w2, enc_b2, codebook,
           dec_w1, dec_b1, dec_w2, dec_b2):
    xn = jnp.transpose(x, (0, 2, 3, 1))
    xp = jnp.pad(xn, ((0, 0), (1, 1), (1, 1), (0, 0)))
    p1p = jnp.stack(
        [jnp.concatenate(
            [xp[:, 2 * py + kh:2 * py + kh + 29:4,
                2 * px + kw:2 * px + kw + 29:4, :]
             for kh in range(4) for kw in range(4)], axis=-1)
         for py in range(2) for px in range(2)], axis=1)

    w1r = enc_w1.transpose(2, 3, 1, 0).reshape(48, HID)
    w2k = enc_w2.transpose(2, 3, 1, 0)
    w2c = jnp.stack(
        [jnp.concatenate(
            [w2k[(1 - pr) + 2 * oy, (1 - pc) + 2 * ox]
             for oy in range(2) for ox in range(2)], axis=0)
         for pr in range(2) for pc in range(2)], axis=0)
    d1t = jnp.pad(dec_w1.transpose(1, 2, 3, 0).reshape(D, 16 * HID),
                  ((0, D), (0, 0)))
    d2k = dec_w2.transpose(2, 3, 1, 0)
    d2a = jnp.stack(
        [jnp.pad(
            jnp.concatenate(
                [d2k[(ty & 1) + 2 * i, (tx & 1) + 2 * j]
                 for i in range(2) for j in range(2)], axis=0),
            ((0, 0), (3 * (ty * 4 + tx), 45 - 3 * (ty * 4 + tx))))
         for ty in range(4) for tx in range(4)], axis=0)
    db2t = jnp.tile(dec_b2, 16)

    cb128 = jnp.pad(codebook, ((0, 0), (0, D)))
    idx = _run_enc(p1p, w1r, enc_b1, w2c, enc_b2, codebook.T)
    q = _run_gather(cb128, idx.reshape(ROWS))
    res48 = _run_dec(q, d1t, dec_b1, d2a, db2t)

    st = res48.reshape(B_TOT, 8, 8, 4, 4, 3)
    return st.transpose(0, 5, 1, 3, 2, 4).reshape(B_TOT, 3, 32, 32)


# speedup vs baseline: 1.9081x; 1.0132x over previous
import jax
import jax.numpy as jnp
from jax.experimental import pallas as pl
from jax.experimental.pallas import tpu as pltpu
from jax.experimental.pallas import tpu_sc as plsc

G = 32
B_TOT = 256
K = 1024
D = 64
HID = 128

NC, NS = 2, 16
NW = NC * NS
ROWS = B_TOT * 64
BPW = ROWS // NW


def _mm(a, b):
    return jax.lax.dot_general(a, b, (((1,), (0,)), ((), ())),
                               preferred_element_type=jnp.float32)


def _enc_block(p1_ref, w1_ref, b1_ref, w2c_ref, b2_ref, cbt_ref, idx_ref):
    g = p1_ref.shape[0]
    f32 = jnp.float32

    p1 = p1_ref[...]
    m = _mm(p1.reshape(g * 4 * 64, 48), w1_ref[...])
    h1 = jax.nn.relu(m + b1_ref[...][None, :]).reshape(g, 4, 8, 8, HID)

    h1p = {}
    for py in range(2):
        for px in range(2):
            h1p[(py, px)] = jnp.pad(
                h1[:, py * 2 + px],
                ((0, 0), (py, 1 - py), (px, 1 - px), (0, 0)))

    w2c = w2c_ref[...]
    e = jnp.zeros((g * 64, D), f32)
    for pr in range(2):
        for pc in range(2):
            a = h1p[(pr, pc)]
            cat = jnp.concatenate(
                [a[:, oy:oy + 8, ox:ox + 8, :]
                 for oy in range(2) for ox in range(2)], axis=-1)
            e = e + _mm(cat.reshape(g * 64, 4 * HID), w2c[pr * 2 + pc])
    e = e + b2_ref[...][None, :]

    cbt = cbt_ref[...]
    cn = jnp.sum(cbt * cbt, axis=0)
    scores = cn[None, :] - 2.0 * _mm(e, cbt)
    idx_ref[...] = jnp.argmin(scores, axis=1).astype(jnp.int32).reshape(g, 64)


SC_STAGE = K // NS


def _sc_gather_body(table_ref, idx_ref, out_ref, tspm, stage_v, idx_v,
                    rows_v, sem):
    sid = jax.lax.axis_index("s")
    wid = sid * NC + jax.lax.axis_index("c")
    pltpu.sync_copy(table_ref.at[pl.ds(sid * SC_STAGE, SC_STAGE)], stage_v)
    pltpu.sync_copy(stage_v, tspm.at[pl.ds(sid * SC_STAGE, SC_STAGE)])
    plsc.subcore_barrier()
    base = wid * BPW
    pltpu.sync_copy(idx_ref.at[pl.ds(base, BPW)], idx_v)
    pltpu.async_copy(tspm.at[idx_v], rows_v, sem).wait()
    pltpu.sync_copy(rows_v, out_ref.at[pl.ds(base, BPW)])


def _dec_block(q_ref, d1t_ref, db1_ref, d2a_ref, db2_ref, out_ref):
    g = q_ref.shape[0] // 64
    q = q_ref[...]

    d1t = d1t_ref[...]
    db1 = db1_ref[...]
    big = _mm(q, d1t)
    pp1 = [jnp.pad(big[:, HID * t:HID * (t + 1)].reshape(g, 8, 8, HID),
                   ((0, 0), (1, 1), (1, 1), (0, 0)))
           for t in range(16)]
    h2 = {}
    for R in range(2):
        for S in range(2):
            acc = None
            for i in range(2):
                sy = R - 1 + i
                for j in range(2):
                    sx = S - 1 + j
                    t = (R + 2 * i) * 4 + (S + 2 * j)
                    sl = pp1[t][:, 1 + sy:9 + sy, 1 + sx:9 + sx, :]
                    acc = sl if acc is None else acc + sl
            h2[(R, S)] = jax.nn.relu(acc + db1[None, None, None, :])

    d2c = d2a_ref[...]
    db2 = db2_ref[...]
    h2p = {k: jnp.pad(v, ((0, 0), (1, 1), (1, 1), (0, 0)))
           for k, v in h2.items()}
    acc48 = None
    for ty in range(4):
        R, apar = ty & 1, ty >> 1
        for tx in range(4):
            S, bpar = tx & 1, tx >> 1
            parts = []
            for i in range(2):
                cy = apar + R - 1 + i
                ry, sy = cy & 1, cy >> 1
                for j in range(2):
                    cx = bpar + S - 1 + j
                    rx, sx = cx & 1, cx >> 1
                    parts.append(h2p[(ry, rx)][:, 1 + sy:9 + sy,
                                               1 + sx:9 + sx, :])
            cat = jnp.concatenate(parts, axis=-1)
            res = _mm(cat.reshape(g * 64, 4 * HID), d2c[ty * 4 + tx])
            acc48 = res if acc48 is None else acc48 + res
    out_ref[...] = (acc48 + db2[None, :]).reshape(g, 8, 8, 48)


def _run_enc(p1p, w1r, b1, w2c, b2, cbt, *, interpret=False):
    grid = (B_TOT // G,)
    full = lambda a: pl.BlockSpec(a.shape, lambda i: (0,) * a.ndim)
    return pl.pallas_call(
        _enc_block,
        grid=grid,
        in_specs=[pl.BlockSpec((G, 4, 8, 8, 48), lambda i: (i, 0, 0, 0, 0)),
                  full(w1r), full(b1), full(w2c), full(b2), full(cbt)],
        out_specs=pl.BlockSpec((G, 64), lambda i: (i, 0)),
        out_shape=jax.ShapeDtypeStruct((B_TOT, 64), jnp.int32),
        interpret=interpret,
    )(p1p, w1r, b1, w2c, b2, cbt)


def _run_gather(cb128, idx_flat):
    mesh = plsc.VectorSubcoreMesh(core_axis_name="c", subcore_axis_name="s")
    return pl.kernel(
        _sc_gather_body,
        out_type=jax.ShapeDtypeStruct((ROWS, 2 * D), jnp.float32),
        mesh=mesh,
        scratch_types=[
            pltpu.VMEM_SHARED((K, 2 * D), jnp.float32),
            pltpu.VMEM((SC_STAGE, 2 * D), jnp.float32),
            pltpu.VMEM((BPW,), jnp.int32),
            pltpu.VMEM((BPW, 2 * D), jnp.float32),
            pltpu.SemaphoreType.DMA,
        ],
    )(cb128, idx_flat)


def _run_dec(q, d1t, db1, d2a, db2t, *, interpret=False):
    grid = (B_TOT // G,)
    full = lambda a: pl.BlockSpec(a.shape, lambda i: (0,) * a.ndim)
    return pl.pallas_call(
        _dec_block,
        grid=grid,
        in_specs=[pl.BlockSpec((G * 64, 2 * D), lambda i: (i, 0)),
                  full(d1t), full(db1), full(d2a), full(db2t)],
        out_specs=pl.BlockSpec((G, 8, 8, 48), lambda i: (i, 0, 0, 0)),
        out_shape=jax.ShapeDtypeStruct((B_TOT, 8, 8, 48), jnp.float32),
        interpret=interpret,
    )(q, d1t, db1, d2a, db2t)


def kernel(x, enc_w1, enc_b1, enc_w2, enc_b2, codebook,
           dec_w1, dec_b1, dec_w2, dec_b2):
    xn = jnp.transpose(x, (0, 2, 3, 1))
    xp = jnp.pad(xn, ((0, 0), (1, 1), (1, 1), (0, 0)))
    p1p = jnp.stack(
        [jnp.concatenate(
            [xp[:, 2 * py + kh:2 * py + kh + 29:4,
                2 * px + kw:2 * px + kw + 29:4, :]
             for kh in range(4) for kw in range(4)], axis=-1)
         for py in range(2) for px in range(2)], axis=1)

    w1r = enc_w1.transpose(2, 3, 1, 0).reshape(48, HID)
    w2k = enc_w2.transpose(2, 3, 1, 0)
    w2c = jnp.stack(
        [jnp.concatenate(
            [w2k[(1 - pr) + 2 * oy, (1 - pc) + 2 * ox]
             for oy in range(2) for ox in range(2)], axis=0)
         for pr in range(2) for pc in range(2)], axis=0)
    d1t = jnp.pad(dec_w1.transpose(1, 2, 3, 0).reshape(D, 16 * HID),
                  ((0, D), (0, 0)))
    d2k = dec_w2.transpose(2, 3, 1, 0)
    d2a = jnp.stack(
        [jnp.pad(
            jnp.concatenate(
                [d2k[(ty & 1) + 2 * i, (tx & 1) + 2 * j]
                 for i in range(2) for j in range(2)], axis=0),
            ((0, 0), (3 * (ty * 4 + tx), 45 - 3 * (ty * 4 + tx))))
         for ty in range(4) for tx in range(4)], axis=0)
    db2t = jnp.tile(dec_b2, 16)

    cb128 = jnp.pad(codebook, ((0, 0), (0, D)))
    idx = _run_enc(p1p, w1r, enc_b1, w2c, enc_b2, codebook.T)
    q = _run_gather(cb128, idx.reshape(ROWS))
    res48 = _run_dec(q, d1t, dec_b1, d2a, db2t)

    st = res48.reshape(B_TOT, 8, 8, 4, 4, 3)
    return st.transpose(0, 5, 1, 3, 2, 4).reshape(B_TOT, 3, 32, 32)
